# baseline (speedup 1.0000x reference)
import functools

import jax
import jax.numpy as jnp
from jax.experimental import pallas as pl
from jax.experimental.pallas import tpu as pltpu


def _fused_kernel(seq1_ref, seq2_ref, w_ref, adj_ref, b_ref, a_ref,
                  h1_ref, h2_ref, c_ref, xw_ref, *, nh, num_blocks, n_rows):
    i = pl.program_id(0)

    @pl.when(i == 0)
    def _():
        w = w_ref[...]
        xw_ref[:, :nh] = jnp.dot(seq1_ref[...], w,
                                 preferred_element_type=jnp.float32)
        xw_ref[:, nh:] = jnp.dot(seq2_ref[...], w,
                                 preferred_element_type=jnp.float32)
        c_ref[...] = jnp.zeros_like(c_ref)

    y = jnp.dot(adj_ref[...], xw_ref[...], preferred_element_type=jnp.float32)
    b = b_ref[...]
    a = a_ref[...]
    h1 = y[:, :nh] + b
    h2 = y[:, nh:] + b
    h1 = jnp.where(h1 >= 0, h1, a * h1)
    h2 = jnp.where(h2 >= 0, h2, a * h2)
    h1_ref[...] = h1
    h2_ref[...] = h2
    c_ref[...] += jnp.sum(h1, axis=0, keepdims=True)

    @pl.when(i == num_blocks - 1)
    def _():
        c_ref[...] = c_ref[...] * (1.0 / n_rows)


def _pick_block(n):
    for bm in (200, 400, 80, 40, 16, 8):
        if n % bm == 0:
            return bm
    return n


def kernel(seq1, seq2, adj, W, b, prelu_a):
    n, nin = seq1.shape
    nh = W.shape[1]
    bm = _pick_block(n)
    num_blocks = n // bm
    b2 = jnp.broadcast_to(b.reshape(1, nh), (1, nh))
    a2 = jnp.broadcast_to(prelu_a.reshape(1, 1), (1, nh))
    h1, h2, c = pl.pallas_call(
        functools.partial(_fused_kernel, nh=nh, num_blocks=num_blocks,
                          n_rows=n),
        grid=(num_blocks,),
        in_specs=[
            pl.BlockSpec((n, nin), lambda i: (0, 0)),
            pl.BlockSpec((n, nin), lambda i: (0, 0)),
            pl.BlockSpec((nin, nh), lambda i: (0, 0)),
            pl.BlockSpec((bm, n), lambda i: (i, 0)),
            pl.BlockSpec((1, nh), lambda i: (0, 0)),
            pl.BlockSpec((1, nh), lambda i: (0, 0)),
        ],
        out_specs=[
            pl.BlockSpec((bm, nh), lambda i: (i, 0)),
            pl.BlockSpec((bm, nh), lambda i: (i, 0)),
            pl.BlockSpec((1, nh), lambda i: (0, 0)),
        ],
        out_shape=[
            jax.ShapeDtypeStruct((n, nh), jnp.float32),
            jax.ShapeDtypeStruct((n, nh), jnp.float32),
            jax.ShapeDtypeStruct((1, nh), jnp.float32),
        ],
        scratch_shapes=[pltpu.VMEM((n, 2 * nh), jnp.float32)],
    )(seq1, seq2, W, adj, b2, a2)
    return (h1, h2, c)


# speedup vs baseline: 1.0148x; 1.0148x over previous
import functools

import jax
import jax.numpy as jnp
from jax.experimental import pallas as pl
from jax.experimental.pallas import tpu as pltpu


def _fused_kernel(seq1_ref, seq2_ref, w_ref, adj_ref, b_ref, a_ref,
                  h1_ref, h2_ref, c_ref, xw_ref, *, nh, num_blocks, n_rows):
    i = pl.program_id(0)

    @pl.when(i == 0)
    def _():
        w = w_ref[...]
        xw_ref[:, :nh] = jnp.dot(seq1_ref[...], w,
                                 preferred_element_type=jnp.float32)
        xw_ref[:, nh:] = jnp.dot(seq2_ref[...], w,
                                 preferred_element_type=jnp.float32)
        c_ref[...] = jnp.zeros_like(c_ref)

    y = jnp.dot(adj_ref[...], xw_ref[...], preferred_element_type=jnp.float32)
    b = b_ref[...]
    a = a_ref[...]
    h1 = y[:, :nh] + b
    h2 = y[:, nh:] + b
    h1 = jnp.where(h1 >= 0, h1, a * h1)
    h2 = jnp.where(h2 >= 0, h2, a * h2)
    h1_ref[...] = h1
    h2_ref[...] = h2
    c_ref[...] += jnp.sum(h1, axis=0, keepdims=True)

    @pl.when(i == num_blocks - 1)
    def _():
        c_ref[...] = c_ref[...] * (1.0 / n_rows)


def _pick_block(n):
    for bm in (400, 200, 80, 40, 16, 8):
        if n % bm == 0:
            return bm
    return n


def kernel(seq1, seq2, adj, W, b, prelu_a):
    n, nin = seq1.shape
    nh = W.shape[1]
    bm = _pick_block(n)
    num_blocks = n // bm
    b2 = jnp.broadcast_to(b.reshape(1, nh), (1, nh))
    a2 = jnp.broadcast_to(prelu_a.reshape(1, 1), (1, nh))
    h1, h2, c = pl.pallas_call(
        functools.partial(_fused_kernel, nh=nh, num_blocks=num_blocks,
                          n_rows=n),
        grid=(num_blocks,),
        in_specs=[
            pl.BlockSpec((n, nin), lambda i: (0, 0)),
            pl.BlockSpec((n, nin), lambda i: (0, 0)),
            pl.BlockSpec((nin, nh), lambda i: (0, 0)),
            pl.BlockSpec((bm, n), lambda i: (i, 0)),
            pl.BlockSpec((1, nh), lambda i: (0, 0)),
            pl.BlockSpec((1, nh), lambda i: (0, 0)),
        ],
        out_specs=[
            pl.BlockSpec((bm, nh), lambda i: (i, 0)),
            pl.BlockSpec((bm, nh), lambda i: (i, 0)),
            pl.BlockSpec((1, nh), lambda i: (0, 0)),
        ],
        out_shape=[
            jax.ShapeDtypeStruct((n, nh), jnp.float32),
            jax.ShapeDtypeStruct((n, nh), jnp.float32),
            jax.ShapeDtypeStruct((1, nh), jnp.float32),
        ],
        scratch_shapes=[pltpu.VMEM((n, 2 * nh), jnp.float32)],
    )(seq1, seq2, W, adj, b2, a2)
    return (h1, h2, c)
